# Initial kernel scaffold; baseline (speedup 1.0000x reference)
import jax
import jax.numpy as jnp
from jax.experimental import pallas as pl


def kernel(pos, edge_index):
    raise NotImplementedError("write your pallas kernel here")


# speedup vs baseline: 4.9706x; 4.9706x over previous
import functools

import jax
import jax.numpy as jnp
import numpy as np
from jax import lax
from jax.experimental import pallas as pl
from jax.experimental.pallas import tpu as pltpu
from jax.experimental.pallas import tpu_sc as plsc

N_NODES = 50000
N_EDGES = 1600000
NUM_BASIS = 32
CUTOFF = 8.0

NW = 32
PER_W = N_EDGES // NW
CH = 2000
NCHUNK = PER_W // CH
GRP = CH // 16


def _sc_d2_kernel(px_hbm, py_hbm, pz_hbm, src_hbm, dst_hbm, out_hbm,
                  sidx, didx, sx, sy, sz, tx, ty, tz, outv, sem):
    wid = lax.axis_index("s") * 2 + lax.axis_index("c")
    base = wid * PER_W

    def chunk_body(c, carry):
        off = base + c * CH
        pltpu.sync_copy(src_hbm.at[pl.ds(off, CH)], sidx)
        pltpu.sync_copy(dst_hbm.at[pl.ds(off, CH)], didx)
        cps = [
            pltpu.async_copy(px_hbm.at[sidx], sx, sem),
            pltpu.async_copy(py_hbm.at[sidx], sy, sem),
            pltpu.async_copy(pz_hbm.at[sidx], sz, sem),
            pltpu.async_copy(px_hbm.at[didx], tx, sem),
            pltpu.async_copy(py_hbm.at[didx], ty, sem),
            pltpu.async_copy(pz_hbm.at[didx], tz, sem),
        ]
        for cp in cps:
            cp.wait()

        def grp_body(g, carry2):
            sl = pl.ds(g * 16, 16)
            vx = sx[sl] - tx[sl]
            vy = sy[sl] - ty[sl]
            vz = sz[sl] - tz[sl]
            outv[sl] = vx * vx + vy * vy + vz * vz
            return carry2

        lax.fori_loop(0, GRP, grp_body, 0, unroll=4)
        pltpu.sync_copy(outv, out_hbm.at[pl.ds(off, CH)])
        return carry

    lax.fori_loop(0, NCHUNK, chunk_body, 0)


@jax.jit
def _sc_d2(px, py, pz, src, dst):
    mesh = plsc.VectorSubcoreMesh(core_axis_name="c", subcore_axis_name="s")
    f = functools.partial(
        pl.kernel,
        mesh=mesh,
        out_type=jax.ShapeDtypeStruct((N_EDGES,), jnp.float32),
        scratch_types=[
            pltpu.VMEM((CH,), jnp.int32),
            pltpu.VMEM((CH,), jnp.int32),
            pltpu.VMEM((CH,), jnp.float32),
            pltpu.VMEM((CH,), jnp.float32),
            pltpu.VMEM((CH,), jnp.float32),
            pltpu.VMEM((CH,), jnp.float32),
            pltpu.VMEM((CH,), jnp.float32),
            pltpu.VMEM((CH,), jnp.float32),
            pltpu.VMEM((CH,), jnp.float32),
            pltpu.SemaphoreType.DMA,
        ],
    )(_sc_d2_kernel)
    return f(px, py, pz, src, dst)


_OFFSETS = np.linspace(0.0, CUTOFF, NUM_BASIS, dtype=np.float32)
_SPACING = float(_OFFSETS[1] - _OFFSETS[0])
_COEFF = float(-0.5 / (_OFFSETS[1] - _OFFSETS[0]) ** 2)

TC_BR = 16
TC_EDGES = TC_BR * 128
TC_ROWS = N_EDGES // 128


def _tc_expand_kernel(d2_ref, out_ref):
    d2b = d2_ref[...]
    dist = jnp.sqrt(d2b)
    u = dist * (np.pi / CUTOFF)
    fc = 0.5 * (jnp.cos(u) + 1.0)
    fc = jnp.where(dist < CUTOFF, fc, 0.0)
    offs = lax.broadcasted_iota(
        jnp.int32, (NUM_BASIS, 1), 0).astype(jnp.float32) * _SPACING
    offs_bc = jnp.broadcast_to(offs, (NUM_BASIS, 128))
    for i in range(TC_BR):
        drow = lax.slice(dist, (i, 0), (i + 1, 128))
        frow = lax.slice(fc, (i, 0), (i + 1, 128))
        dbc = jnp.broadcast_to(drow, (NUM_BASIS, 128))
        t = dbc - offs_bc
        r = jnp.exp(_COEFF * (t * t)) * jnp.broadcast_to(frow,
                                                         (NUM_BASIS, 128))
        out_ref[pl.ds(i * 128, 128), :] = r.T


@jax.jit
def _tc_expand(d2_2d):
    grid = (TC_ROWS + TC_BR - 1) // TC_BR
    return pl.pallas_call(
        _tc_expand_kernel,
        grid=(grid,),
        in_specs=[pl.BlockSpec((TC_BR, 128), lambda i: (i, 0))],
        out_specs=pl.BlockSpec((TC_EDGES, NUM_BASIS), lambda i: (i, 0)),
        out_shape=jax.ShapeDtypeStruct((N_EDGES, NUM_BASIS), jnp.float32),
    )(d2_2d)


def kernel(pos, edge_index):
    px = pos[:, 0]
    py = pos[:, 1]
    pz = pos[:, 2]
    src = edge_index[0]
    dst = edge_index[1]
    d2 = _sc_d2(px, py, pz, src, dst)
    return _tc_expand(d2.reshape(TC_ROWS, 128))


# speedup vs baseline: 6.1869x; 1.2447x over previous
import functools

import jax
import jax.numpy as jnp
import numpy as np
from jax import lax
from jax.experimental import pallas as pl
from jax.experimental.pallas import tpu as pltpu
from jax.experimental.pallas import tpu_sc as plsc

N_NODES = 50000
N_EDGES = 1600000
NUM_BASIS = 32
CUTOFF = 8.0

NW = 32
PER_W = N_EDGES // NW
CH = 2000
NCHUNK = PER_W // CH
GRP = CH // 16


STAGE_SL = N_NODES // 10


def _sc_d2_kernel(px_hbm, py_hbm, pz_hbm, src_hbm, dst_hbm, out_hbm,
                  sidx, didx, sx, sy, sz, tx, ty, tz, outv,
                  pxs, pys, pzs, stg, sem):
    sid = lax.axis_index("s")
    cid = lax.axis_index("c")
    wid = sid * 2 + cid
    base = wid * PER_W

    jobs = [(px_hbm, pxs), (py_hbm, pys), (pz_hbm, pzs)]
    for j in range(30):
        hbm, sp = jobs[j // 10]
        sl = pl.ds((j % 10) * STAGE_SL, STAGE_SL)

        @pl.when(sid == (j % 16))
        def _():
            pltpu.sync_copy(hbm.at[sl], stg)
            pltpu.sync_copy(stg, sp.at[sl])

    plsc.subcore_barrier()

    def chunk_body(c, carry):
        off = base + c * CH
        pltpu.sync_copy(src_hbm.at[pl.ds(off, CH)], sidx)
        pltpu.sync_copy(dst_hbm.at[pl.ds(off, CH)], didx)
        cps = [
            pltpu.async_copy(pxs.at[sidx], sx, sem),
            pltpu.async_copy(pys.at[sidx], sy, sem),
            pltpu.async_copy(pzs.at[sidx], sz, sem),
            pltpu.async_copy(pxs.at[didx], tx, sem),
            pltpu.async_copy(pys.at[didx], ty, sem),
            pltpu.async_copy(pzs.at[didx], tz, sem),
        ]
        for cp in cps:
            cp.wait()

        def grp_body(g, carry2):
            sl = pl.ds(g * 16, 16)
            vx = sx[sl] - tx[sl]
            vy = sy[sl] - ty[sl]
            vz = sz[sl] - tz[sl]
            outv[sl] = vx * vx + vy * vy + vz * vz
            return carry2

        lax.fori_loop(0, GRP, grp_body, 0, unroll=4)
        pltpu.sync_copy(outv, out_hbm.at[pl.ds(off, CH)])
        return carry

    lax.fori_loop(0, NCHUNK, chunk_body, 0)


@jax.jit
def _sc_d2(px, py, pz, src, dst):
    mesh = plsc.VectorSubcoreMesh(core_axis_name="c", subcore_axis_name="s")
    f = functools.partial(
        pl.kernel,
        mesh=mesh,
        out_type=jax.ShapeDtypeStruct((N_EDGES,), jnp.float32),
        scratch_types=[
            pltpu.VMEM((CH,), jnp.int32),
            pltpu.VMEM((CH,), jnp.int32),
            pltpu.VMEM((CH,), jnp.float32),
            pltpu.VMEM((CH,), jnp.float32),
            pltpu.VMEM((CH,), jnp.float32),
            pltpu.VMEM((CH,), jnp.float32),
            pltpu.VMEM((CH,), jnp.float32),
            pltpu.VMEM((CH,), jnp.float32),
            pltpu.VMEM((CH,), jnp.float32),
            pltpu.VMEM_SHARED((N_NODES,), jnp.float32),
            pltpu.VMEM_SHARED((N_NODES,), jnp.float32),
            pltpu.VMEM_SHARED((N_NODES,), jnp.float32),
            pltpu.VMEM((STAGE_SL,), jnp.float32),
            pltpu.SemaphoreType.DMA,
        ],
    )(_sc_d2_kernel)
    return f(px, py, pz, src, dst)


_OFFSETS = np.linspace(0.0, CUTOFF, NUM_BASIS, dtype=np.float32)
_SPACING = float(_OFFSETS[1] - _OFFSETS[0])
_COEFF = float(-0.5 / (_OFFSETS[1] - _OFFSETS[0]) ** 2)

TC_BR = 16
TC_EDGES = TC_BR * 128
TC_ROWS = N_EDGES // 128


def _tc_expand_kernel(d2_ref, out_ref):
    d2b = d2_ref[...]
    dist = jnp.sqrt(d2b)
    u = dist * (np.pi / CUTOFF)
    fc = 0.5 * (jnp.cos(u) + 1.0)
    fc = jnp.where(dist < CUTOFF, fc, 0.0)
    offs = lax.broadcasted_iota(
        jnp.int32, (NUM_BASIS, 1), 0).astype(jnp.float32) * _SPACING
    offs_bc = jnp.broadcast_to(offs, (NUM_BASIS, 128))
    for i in range(TC_BR):
        drow = lax.slice(dist, (i, 0), (i + 1, 128))
        frow = lax.slice(fc, (i, 0), (i + 1, 128))
        dbc = jnp.broadcast_to(drow, (NUM_BASIS, 128))
        t = dbc - offs_bc
        r = jnp.exp(_COEFF * (t * t)) * jnp.broadcast_to(frow,
                                                         (NUM_BASIS, 128))
        out_ref[pl.ds(i * 128, 128), :] = r.T


@jax.jit
def _tc_expand(d2_2d):
    grid = (TC_ROWS + TC_BR - 1) // TC_BR
    return pl.pallas_call(
        _tc_expand_kernel,
        grid=(grid,),
        in_specs=[pl.BlockSpec((TC_BR, 128), lambda i: (i, 0))],
        out_specs=pl.BlockSpec((TC_EDGES, NUM_BASIS), lambda i: (i, 0)),
        out_shape=jax.ShapeDtypeStruct((N_EDGES, NUM_BASIS), jnp.float32),
    )(d2_2d)


def kernel(pos, edge_index):
    px = pos[:, 0]
    py = pos[:, 1]
    pz = pos[:, 2]
    src = edge_index[0]
    dst = edge_index[1]
    d2 = _sc_d2(px, py, pz, src, dst)
    return _tc_expand(d2.reshape(TC_ROWS, 128))


# speedup vs baseline: 22.7512x; 3.6773x over previous
import functools

import jax
import jax.numpy as jnp
import numpy as np
from jax import lax
from jax.experimental import pallas as pl
from jax.experimental.pallas import tpu as pltpu
from jax.experimental.pallas import tpu_sc as plsc

N_NODES = 50000
N_EDGES = 1600000
NUM_BASIS = 32
CUTOFF = 8.0

NW = 32
PER_W = N_EDGES // NW
CH = 2000
NCHUNK = PER_W // CH
GRP = CH // 16

STAGE_SL = N_NODES // 10


def _sc_d2_kernel(px_hbm, py_hbm, pz_hbm, src_hbm, dst_hbm, out_hbm,
                  sidx, didx, sx, sy, sz, tx, ty, tz, outv,
                  pxs, pys, pzs, stg, sem):
    sid = lax.axis_index("s")
    cid = lax.axis_index("c")
    wid = sid * 2 + cid
    base = wid * PER_W

    jobs = [(px_hbm, pxs), (py_hbm, pys), (pz_hbm, pzs)]
    for j in range(30):
        hbm, sp = jobs[j // 10]
        sl = pl.ds((j % 10) * STAGE_SL, STAGE_SL)

        @pl.when(sid == (j % 16))
        def _():
            pltpu.sync_copy(hbm.at[sl], stg)
            pltpu.sync_copy(stg, sp.at[sl])

    plsc.subcore_barrier()

    def chunk_body(c, carry):
        off = base + c * CH
        pltpu.sync_copy(src_hbm.at[pl.ds(off, CH)], sidx)
        pltpu.sync_copy(dst_hbm.at[pl.ds(off, CH)], didx)
        cps = [
            pltpu.async_copy(pxs.at[sidx], sx, sem),
            pltpu.async_copy(pys.at[sidx], sy, sem),
            pltpu.async_copy(pzs.at[sidx], sz, sem),
            pltpu.async_copy(pxs.at[didx], tx, sem),
            pltpu.async_copy(pys.at[didx], ty, sem),
            pltpu.async_copy(pzs.at[didx], tz, sem),
        ]
        for cp in cps:
            cp.wait()

        def grp_body(g, carry2):
            sl = pl.ds(g * 16, 16)
            vx = sx[sl] - tx[sl]
            vy = sy[sl] - ty[sl]
            vz = sz[sl] - tz[sl]
            outv[sl] = vx * vx + vy * vy + vz * vz
            return carry2

        lax.fori_loop(0, GRP, grp_body, 0, unroll=4)
        pltpu.sync_copy(outv, out_hbm.at[pl.ds(off, CH)])
        return carry

    lax.fori_loop(0, NCHUNK, chunk_body, 0)


@jax.jit
def _sc_d2(px, py, pz, src, dst):
    mesh = plsc.VectorSubcoreMesh(core_axis_name="c", subcore_axis_name="s")
    f = functools.partial(
        pl.kernel,
        mesh=mesh,
        out_type=jax.ShapeDtypeStruct((N_EDGES,), jnp.float32),
        scratch_types=[
            pltpu.VMEM((CH,), jnp.int32),
            pltpu.VMEM((CH,), jnp.int32),
            pltpu.VMEM((CH,), jnp.float32),
            pltpu.VMEM((CH,), jnp.float32),
            pltpu.VMEM((CH,), jnp.float32),
            pltpu.VMEM((CH,), jnp.float32),
            pltpu.VMEM((CH,), jnp.float32),
            pltpu.VMEM((CH,), jnp.float32),
            pltpu.VMEM((CH,), jnp.float32),
            pltpu.VMEM_SHARED((N_NODES,), jnp.float32),
            pltpu.VMEM_SHARED((N_NODES,), jnp.float32),
            pltpu.VMEM_SHARED((N_NODES,), jnp.float32),
            pltpu.VMEM((STAGE_SL,), jnp.float32),
            pltpu.SemaphoreType.DMA,
        ],
    )(_sc_d2_kernel)
    return f(px, py, pz, src, dst)


_OFFSETS = np.linspace(0.0, CUTOFF, NUM_BASIS, dtype=np.float32)
_SPACING = float(_OFFSETS[1] - _OFFSETS[0])
_COEFF = float(-0.5 / (_OFFSETS[1] - _OFFSETS[0]) ** 2)

TC_BE = 16384


def _tc_expand_kernel(d2_ref, out_ref):
    d2v = d2_ref[...]
    dist = jnp.sqrt(d2v)
    u = dist * (np.pi / CUTOFF)
    fc = 0.5 * (jnp.cos(u) + 1.0)
    fc = jnp.where(dist < CUTOFF, fc, 0.0)
    db = jnp.broadcast_to(dist[None, :], (NUM_BASIS, TC_BE))
    fcb = jnp.broadcast_to(fc[None, :], (NUM_BASIS, TC_BE))
    offs = lax.broadcasted_iota(
        jnp.int32, (NUM_BASIS, 1), 0).astype(jnp.float32) * _SPACING
    offs_bc = jnp.broadcast_to(offs, (NUM_BASIS, TC_BE))
    t = db - offs_bc
    out_ref[...] = jnp.exp(_COEFF * (t * t)) * fcb


@jax.jit
def _tc_expand(d2):
    grid = ((N_EDGES + TC_BE - 1) // TC_BE,)
    out_t = pl.pallas_call(
        _tc_expand_kernel,
        grid=grid,
        in_specs=[pl.BlockSpec((TC_BE,), lambda i: (i,))],
        out_specs=pl.BlockSpec((NUM_BASIS, TC_BE), lambda i: (0, i)),
        out_shape=jax.ShapeDtypeStruct((NUM_BASIS, N_EDGES), jnp.float32),
    )(d2)
    return out_t.T


def kernel(pos, edge_index):
    px = pos[:, 0]
    py = pos[:, 1]
    pz = pos[:, 2]
    d2 = _sc_d2(px, py, pz, edge_index[0], edge_index[1])
    return _tc_expand(d2)


# speedup vs baseline: 25.7478x; 1.1317x over previous
import functools

import jax
import jax.numpy as jnp
import numpy as np
from jax import lax
from jax.experimental import pallas as pl
from jax.experimental.pallas import tpu as pltpu
from jax.experimental.pallas import tpu_sc as plsc

N_NODES = 50000
N_EDGES = 1600000
NUM_BASIS = 32
CUTOFF = 8.0

NW = 32
PER_W = N_EDGES // NW
CH = 2000
NCHUNK = PER_W // CH
GRP = CH // 16

def _sc_d2_kernel(px_hbm, py_hbm, pz_hbm, src_hbm, dst_hbm, out_hbm,
                  xt, yt, sidx0, sidx1, didx0, didx1, outv0, outv1,
                  sem_i0, sem_i1, sem_b0, sem_b1):
    sid = lax.axis_index("s")
    cid = lax.axis_index("c")
    wid = sid * 2 + cid
    base = wid * PER_W
    sidx = (sidx0, sidx1)
    didx = (didx0, didx1)
    outv = (outv0, outv1)
    sem_i = (sem_i0, sem_i1)
    sem_b = (sem_b0, sem_b1)

    def start_idx(c):
        off = base + c * CH
        return (
            pltpu.async_copy(src_hbm.at[pl.ds(off, CH)], sidx[c % 2],
                             sem_i[c % 2]),
            pltpu.async_copy(dst_hbm.at[pl.ds(off, CH)], didx[c % 2],
                             sem_i[c % 2]),
        )

    pltpu.sync_copy(px_hbm, xt)
    pltpu.sync_copy(py_hbm, yt)

    out_cps = [None, None]
    idx_cp = start_idx(0)
    for c in range(NCHUNK):
        b = c % 2
        nxt_cp = start_idx(c + 1) if c + 1 < NCHUNK else None
        for cp in idx_cp:
            cp.wait()
        if out_cps[b] is not None:
            out_cps[b].wait()
        sb, db, ob = sidx[b], didx[b], outv[b]

        def grp_body(g, carry, sb=sb, db=db, ob=ob):
            sl = pl.ds(g * 16, 16)
            si = sb[sl]
            di = db[sl]
            vx = plsc.load_gather(xt, [si]) - plsc.load_gather(xt, [di])
            vy = plsc.load_gather(yt, [si]) - plsc.load_gather(yt, [di])
            ob[sl] = vx * vx + vy * vy
            return carry

        lax.fori_loop(0, GRP, grp_body, 0, unroll=4)
        off = base + c * CH
        out_cps[b] = pltpu.async_copy(ob, out_hbm.at[pl.ds(off, CH)],
                                      sem_b[b])
        idx_cp = nxt_cp
    for cp in out_cps:
        if cp is not None:
            cp.wait()

    pltpu.sync_copy(pz_hbm, xt)

    in_cps = [None, None]
    out_cps = [None, None]
    idx_cp = start_idx(0)
    in_cps[0] = pltpu.async_copy(out_hbm.at[pl.ds(base, CH)], outv0, sem_b0)
    for c in range(NCHUNK):
        b = c % 2
        if c + 1 < NCHUNK:
            nxt_cp = start_idx(c + 1)
            nb = (c + 1) % 2
            if out_cps[nb] is not None:
                out_cps[nb].wait()
            in_cps[nb] = pltpu.async_copy(
                out_hbm.at[pl.ds(base + (c + 1) * CH, CH)], outv[nb],
                sem_b[nb])
        else:
            nxt_cp = None
        for cp in idx_cp:
            cp.wait()
        in_cps[b].wait()
        sb, db, ob = sidx[b], didx[b], outv[b]

        def grp_body(g, carry, sb=sb, db=db, ob=ob):
            sl = pl.ds(g * 16, 16)
            vz = plsc.load_gather(xt, [sb[sl]]) - plsc.load_gather(xt, [db[sl]])
            ob[sl] = ob[sl] + vz * vz
            return carry

        lax.fori_loop(0, GRP, grp_body, 0, unroll=4)
        off = base + c * CH
        out_cps[b] = pltpu.async_copy(ob, out_hbm.at[pl.ds(off, CH)],
                                      sem_b[b])
        idx_cp = nxt_cp
    for cp in out_cps:
        if cp is not None:
            cp.wait()


@jax.jit
def _sc_d2(px, py, pz, src, dst):
    mesh = plsc.VectorSubcoreMesh(core_axis_name="c", subcore_axis_name="s")
    f = functools.partial(
        pl.kernel,
        mesh=mesh,
        compiler_params=pltpu.CompilerParams(needs_layout_passes=False),
        out_type=jax.ShapeDtypeStruct((N_EDGES,), jnp.float32),
        scratch_types=[
            pltpu.VMEM((N_NODES,), jnp.float32),
            pltpu.VMEM((N_NODES,), jnp.float32),
            pltpu.VMEM((CH,), jnp.int32),
            pltpu.VMEM((CH,), jnp.int32),
            pltpu.VMEM((CH,), jnp.int32),
            pltpu.VMEM((CH,), jnp.int32),
            pltpu.VMEM((CH,), jnp.float32),
            pltpu.VMEM((CH,), jnp.float32),
            pltpu.SemaphoreType.DMA,
            pltpu.SemaphoreType.DMA,
            pltpu.SemaphoreType.DMA,
            pltpu.SemaphoreType.DMA,
        ],
    )(_sc_d2_kernel)
    return f(px, py, pz, src, dst)


_OFFSETS = np.linspace(0.0, CUTOFF, NUM_BASIS, dtype=np.float32)
_SPACING = float(_OFFSETS[1] - _OFFSETS[0])
_COEFF = float(-0.5 / (_OFFSETS[1] - _OFFSETS[0]) ** 2)

TC_BE = 16384


def _tc_expand_kernel(d2_ref, out_ref):
    d2v = d2_ref[...]
    dist = jnp.sqrt(d2v)
    u = dist * (np.pi / CUTOFF)
    fc = 0.5 * (jnp.cos(u) + 1.0)
    fc = jnp.where(dist < CUTOFF, fc, 0.0)
    db = jnp.broadcast_to(dist[None, :], (NUM_BASIS, TC_BE))
    fcb = jnp.broadcast_to(fc[None, :], (NUM_BASIS, TC_BE))
    offs = lax.broadcasted_iota(
        jnp.int32, (NUM_BASIS, 1), 0).astype(jnp.float32) * _SPACING
    offs_bc = jnp.broadcast_to(offs, (NUM_BASIS, TC_BE))
    t = db - offs_bc
    out_ref[...] = jnp.exp(_COEFF * (t * t)) * fcb


@jax.jit
def _tc_expand(d2):
    grid = ((N_EDGES + TC_BE - 1) // TC_BE,)
    out_t = pl.pallas_call(
        _tc_expand_kernel,
        grid=grid,
        in_specs=[pl.BlockSpec((TC_BE,), lambda i: (i,))],
        out_specs=pl.BlockSpec((NUM_BASIS, TC_BE), lambda i: (0, i)),
        out_shape=jax.ShapeDtypeStruct((NUM_BASIS, N_EDGES), jnp.float32),
    )(d2)
    return out_t.T


def kernel(pos, edge_index):
    px = pos[:, 0]
    py = pos[:, 1]
    pz = pos[:, 2]
    d2 = _sc_d2(px, py, pz, edge_index[0], edge_index[1])
    return _tc_expand(d2)


# speedup vs baseline: 31.5207x; 1.2242x over previous
import functools

import jax
import jax.numpy as jnp
import numpy as np
from jax import lax
from jax.experimental import pallas as pl
from jax.experimental.pallas import tpu as pltpu
from jax.experimental.pallas import tpu_sc as plsc

N_NODES = 50000
N_EDGES = 1600000
NUM_BASIS = 32
CUTOFF = 8.0

NW = 32
PER_W = N_EDGES // NW
CH = 2000
NCHUNK = PER_W // CH
GRP = CH // 16

def _sc_d2_kernel(px_hbm, py_hbm, pz_hbm, src_hbm, dst_hbm, out_hbm,
                  xt, yt, sidx0, sidx1, didx0, didx1, outv0, outv1,
                  sem_i0, sem_i1, sem_b0, sem_b1):
    sid = lax.axis_index("s")
    cid = lax.axis_index("c")
    wid = sid * 2 + cid
    base = wid * PER_W
    sidx = (sidx0, sidx1)
    didx = (didx0, didx1)
    outv = (outv0, outv1)
    sem_i = (sem_i0, sem_i1)
    sem_b = (sem_b0, sem_b1)

    def start_idx(c):
        off = base + c * CH
        return (
            pltpu.async_copy(src_hbm.at[pl.ds(off, CH)], sidx[c % 2],
                             sem_i[c % 2]),
            pltpu.async_copy(dst_hbm.at[pl.ds(off, CH)], didx[c % 2],
                             sem_i[c % 2]),
        )

    pltpu.sync_copy(px_hbm, xt)
    pltpu.sync_copy(py_hbm, yt)

    out_cps = [None, None]
    idx_cp = start_idx(0)
    for c in range(NCHUNK):
        b = c % 2
        nxt_cp = start_idx(c + 1) if c + 1 < NCHUNK else None
        for cp in idx_cp:
            cp.wait()
        if out_cps[b] is not None:
            out_cps[b].wait()
        sb, db, ob = sidx[b], didx[b], outv[b]

        @plsc.parallel_loop(0, GRP, unroll=5)
        def _(g, sb=sb, db=db, ob=ob):
            sl = pl.ds(g * 16, 16)
            si = sb[sl]
            di = db[sl]
            vx = plsc.load_gather(xt, [si]) - plsc.load_gather(xt, [di])
            vy = plsc.load_gather(yt, [si]) - plsc.load_gather(yt, [di])
            ob[sl] = vx * vx + vy * vy

        off = base + c * CH
        out_cps[b] = pltpu.async_copy(ob, out_hbm.at[pl.ds(off, CH)],
                                      sem_b[b])
        idx_cp = nxt_cp
    for cp in out_cps:
        if cp is not None:
            cp.wait()

    pltpu.sync_copy(pz_hbm, xt)

    in_cps = [None, None]
    out_cps = [None, None]
    idx_cp = start_idx(0)
    in_cps[0] = pltpu.async_copy(out_hbm.at[pl.ds(base, CH)], outv0, sem_b0)
    for c in range(NCHUNK):
        b = c % 2
        if c + 1 < NCHUNK:
            nxt_cp = start_idx(c + 1)
            nb = (c + 1) % 2
            if out_cps[nb] is not None:
                out_cps[nb].wait()
            in_cps[nb] = pltpu.async_copy(
                out_hbm.at[pl.ds(base + (c + 1) * CH, CH)], outv[nb],
                sem_b[nb])
        else:
            nxt_cp = None
        for cp in idx_cp:
            cp.wait()
        in_cps[b].wait()
        sb, db, ob = sidx[b], didx[b], outv[b]

        @plsc.parallel_loop(0, GRP, unroll=5)
        def _(g, sb=sb, db=db, ob=ob):
            sl = pl.ds(g * 16, 16)
            vz = plsc.load_gather(xt, [sb[sl]]) - plsc.load_gather(xt, [db[sl]])
            ob[sl] = ob[sl] + vz * vz

        off = base + c * CH
        out_cps[b] = pltpu.async_copy(ob, out_hbm.at[pl.ds(off, CH)],
                                      sem_b[b])
        idx_cp = nxt_cp
    for cp in out_cps:
        if cp is not None:
            cp.wait()


@jax.jit
def _sc_d2(px, py, pz, src, dst):
    mesh = plsc.VectorSubcoreMesh(core_axis_name="c", subcore_axis_name="s")
    f = functools.partial(
        pl.kernel,
        mesh=mesh,
        compiler_params=pltpu.CompilerParams(needs_layout_passes=False),
        out_type=jax.ShapeDtypeStruct((N_EDGES,), jnp.float32),
        scratch_types=[
            pltpu.VMEM((N_NODES,), jnp.float32),
            pltpu.VMEM((N_NODES,), jnp.float32),
            pltpu.VMEM((CH,), jnp.int32),
            pltpu.VMEM((CH,), jnp.int32),
            pltpu.VMEM((CH,), jnp.int32),
            pltpu.VMEM((CH,), jnp.int32),
            pltpu.VMEM((CH,), jnp.float32),
            pltpu.VMEM((CH,), jnp.float32),
            pltpu.SemaphoreType.DMA,
            pltpu.SemaphoreType.DMA,
            pltpu.SemaphoreType.DMA,
            pltpu.SemaphoreType.DMA,
        ],
    )(_sc_d2_kernel)
    return f(px, py, pz, src, dst)


_OFFSETS = np.linspace(0.0, CUTOFF, NUM_BASIS, dtype=np.float32)
_SPACING = float(_OFFSETS[1] - _OFFSETS[0])
_COEFF = float(-0.5 / (_OFFSETS[1] - _OFFSETS[0]) ** 2)

TC_BE = 16384


def _tc_expand_kernel(d2_ref, out_ref):
    d2v = d2_ref[...]
    dist = jnp.sqrt(d2v)
    u = dist * (np.pi / CUTOFF)
    fc = 0.5 * (jnp.cos(u) + 1.0)
    fc = jnp.where(dist < CUTOFF, fc, 0.0)
    db = jnp.broadcast_to(dist[None, :], (NUM_BASIS, TC_BE))
    fcb = jnp.broadcast_to(fc[None, :], (NUM_BASIS, TC_BE))
    offs = lax.broadcasted_iota(
        jnp.int32, (NUM_BASIS, 1), 0).astype(jnp.float32) * _SPACING
    offs_bc = jnp.broadcast_to(offs, (NUM_BASIS, TC_BE))
    t = db - offs_bc
    out_ref[...] = jnp.exp(_COEFF * (t * t)) * fcb


@jax.jit
def _tc_expand(d2):
    grid = ((N_EDGES + TC_BE - 1) // TC_BE,)
    out_t = pl.pallas_call(
        _tc_expand_kernel,
        grid=grid,
        in_specs=[pl.BlockSpec((TC_BE,), lambda i: (i,))],
        out_specs=pl.BlockSpec((NUM_BASIS, TC_BE), lambda i: (0, i)),
        out_shape=jax.ShapeDtypeStruct((NUM_BASIS, N_EDGES), jnp.float32),
    )(d2)
    return out_t.T


def kernel(pos, edge_index):
    px = pos[:, 0]
    py = pos[:, 1]
    pz = pos[:, 2]
    d2 = _sc_d2(px, py, pz, edge_index[0], edge_index[1])
    return _tc_expand(d2)


# speedup vs baseline: 40.6463x; 1.2895x over previous
import functools

import jax
import jax.numpy as jnp
import numpy as np
from jax import lax
from jax.experimental import pallas as pl
from jax.experimental.pallas import tpu as pltpu
from jax.experimental.pallas import tpu_sc as plsc

N_NODES = 50000
N_EDGES = 1600000
NUM_BASIS = 32
CUTOFF = 8.0

NW = 32
PER_W = N_EDGES // NW
CH = 2000
NCHUNK = PER_W // CH
GRP = CH // 16

def _sc_d2_kernel(px_hbm, py_hbm, pz_hbm, src_hbm, dst_hbm, out_hbm,
                  xt, yt, sidx0, sidx1, didx0, didx1, outv0, outv1,
                  sem_i0, sem_i1, sem_b0, sem_b1):
    sid = lax.axis_index("s")
    cid = lax.axis_index("c")
    wid = sid * 2 + cid
    base = wid * PER_W
    sidx = (sidx0, sidx1)
    didx = (didx0, didx1)
    outv = (outv0, outv1)
    sem_i = (sem_i0, sem_i1)
    sem_b = (sem_b0, sem_b1)

    def start_idx(c):
        off = base + c * CH
        return (
            pltpu.async_copy(src_hbm.at[pl.ds(off, CH)], sidx[c % 2],
                             sem_i[c % 2]),
            pltpu.async_copy(dst_hbm.at[pl.ds(off, CH)], didx[c % 2],
                             sem_i[c % 2]),
        )

    pltpu.sync_copy(px_hbm, xt)
    pltpu.sync_copy(py_hbm, yt)

    out_cps = [None, None]
    idx_cp = start_idx(0)
    for c in range(NCHUNK):
        b = c % 2
        nxt_cp = start_idx(c + 1) if c + 1 < NCHUNK else None
        for cp in idx_cp:
            cp.wait()
        if out_cps[b] is not None:
            out_cps[b].wait()
        sb, db, ob = sidx[b], didx[b], outv[b]

        @plsc.parallel_loop(0, GRP, unroll=5)
        def _(g, sb=sb, db=db, ob=ob):
            sl = pl.ds(g * 16, 16)
            si = sb[sl]
            di = db[sl]
            vx = plsc.load_gather(xt, [si]) - plsc.load_gather(xt, [di])
            vy = plsc.load_gather(yt, [si]) - plsc.load_gather(yt, [di])
            ob[sl] = vx * vx + vy * vy

        off = base + c * CH
        out_cps[b] = pltpu.async_copy(ob, out_hbm.at[pl.ds(off, CH)],
                                      sem_b[b])
        idx_cp = nxt_cp
    for cp in out_cps:
        if cp is not None:
            cp.wait()

    pltpu.sync_copy(pz_hbm, xt)

    in_cps = [None, None]
    out_cps = [None, None]
    idx_cp = start_idx(0)
    in_cps[0] = pltpu.async_copy(out_hbm.at[pl.ds(base, CH)], outv0, sem_b0)
    for c in range(NCHUNK):
        b = c % 2
        if c + 1 < NCHUNK:
            nxt_cp = start_idx(c + 1)
            nb = (c + 1) % 2
            if out_cps[nb] is not None:
                out_cps[nb].wait()
            in_cps[nb] = pltpu.async_copy(
                out_hbm.at[pl.ds(base + (c + 1) * CH, CH)], outv[nb],
                sem_b[nb])
        else:
            nxt_cp = None
        for cp in idx_cp:
            cp.wait()
        in_cps[b].wait()
        sb, db, ob = sidx[b], didx[b], outv[b]

        @plsc.parallel_loop(0, GRP, unroll=5)
        def _(g, sb=sb, db=db, ob=ob):
            sl = pl.ds(g * 16, 16)
            vz = plsc.load_gather(xt, [sb[sl]]) - plsc.load_gather(xt, [db[sl]])
            ob[sl] = ob[sl] + vz * vz

        off = base + c * CH
        out_cps[b] = pltpu.async_copy(ob, out_hbm.at[pl.ds(off, CH)],
                                      sem_b[b])
        idx_cp = nxt_cp
    for cp in out_cps:
        if cp is not None:
            cp.wait()


@jax.jit
def _sc_d2(px, py, pz, src, dst):
    mesh = plsc.VectorSubcoreMesh(core_axis_name="c", subcore_axis_name="s")
    f = functools.partial(
        pl.kernel,
        mesh=mesh,
        compiler_params=pltpu.CompilerParams(needs_layout_passes=False),
        out_type=jax.ShapeDtypeStruct((N_EDGES,), jnp.float32),
        scratch_types=[
            pltpu.VMEM((N_NODES,), jnp.float32),
            pltpu.VMEM((N_NODES,), jnp.float32),
            pltpu.VMEM((CH,), jnp.int32),
            pltpu.VMEM((CH,), jnp.int32),
            pltpu.VMEM((CH,), jnp.int32),
            pltpu.VMEM((CH,), jnp.int32),
            pltpu.VMEM((CH,), jnp.float32),
            pltpu.VMEM((CH,), jnp.float32),
            pltpu.SemaphoreType.DMA,
            pltpu.SemaphoreType.DMA,
            pltpu.SemaphoreType.DMA,
            pltpu.SemaphoreType.DMA,
        ],
    )(_sc_d2_kernel)
    return f(px, py, pz, src, dst)


_OFFSETS = np.linspace(0.0, CUTOFF, NUM_BASIS, dtype=np.float32)
_SPACING = float(_OFFSETS[1] - _OFFSETS[0])
_COEFF = float(-0.5 / (_OFFSETS[1] - _OFFSETS[0]) ** 2)

DI_BE = 32768


def _deint_kernel(ei_ref, s_ref, d_ref):
    s_ref[...] = ei_ref[0]
    d_ref[...] = ei_ref[1]


@jax.jit
def _deinterleave(ei):
    grid = ((N_EDGES + DI_BE - 1) // DI_BE,)
    return pl.pallas_call(
        _deint_kernel,
        grid=grid,
        in_specs=[pl.BlockSpec((2, DI_BE), lambda i: (0, i))],
        out_specs=[pl.BlockSpec((DI_BE,), lambda i: (i,)),
                   pl.BlockSpec((DI_BE,), lambda i: (i,))],
        out_shape=[jax.ShapeDtypeStruct((N_EDGES,), jnp.int32),
                   jax.ShapeDtypeStruct((N_EDGES,), jnp.int32)],
    )(ei)


TC_BE = 32768


def _tc_expand_kernel(d2_ref, out_ref):
    d2v = d2_ref[...]
    dist = jnp.sqrt(d2v)
    u = dist * (np.pi / CUTOFF)
    fc = 0.5 * (jnp.cos(u) + 1.0)
    fc = jnp.where(dist < CUTOFF, fc, 0.0)
    db = jnp.broadcast_to(dist[None, :], (NUM_BASIS, TC_BE))
    fcb = jnp.broadcast_to(fc[None, :], (NUM_BASIS, TC_BE))
    offs = lax.broadcasted_iota(
        jnp.int32, (NUM_BASIS, 1), 0).astype(jnp.float32) * _SPACING
    offs_bc = jnp.broadcast_to(offs, (NUM_BASIS, TC_BE))
    t = db - offs_bc
    out_ref[...] = jnp.exp(_COEFF * (t * t)) * fcb


@jax.jit
def _tc_expand(d2):
    grid = ((N_EDGES + TC_BE - 1) // TC_BE,)
    out_t = pl.pallas_call(
        _tc_expand_kernel,
        grid=grid,
        in_specs=[pl.BlockSpec((TC_BE,), lambda i: (i,))],
        out_specs=pl.BlockSpec((NUM_BASIS, TC_BE), lambda i: (0, i)),
        out_shape=jax.ShapeDtypeStruct((NUM_BASIS, N_EDGES), jnp.float32),
    )(d2)
    return out_t.T


def kernel(pos, edge_index):
    px = pos[:, 0]
    py = pos[:, 1]
    pz = pos[:, 2]
    src, dst = _deinterleave(edge_index)
    d2 = _sc_d2(px, py, pz, src, dst)
    return _tc_expand(d2)


# speedup vs baseline: 42.4441x; 1.0442x over previous
import functools

import jax
import jax.numpy as jnp
import numpy as np
from jax import lax
from jax.experimental import pallas as pl
from jax.experimental.pallas import tpu as pltpu
from jax.experimental.pallas import tpu_sc as plsc

N_NODES = 50000
N_EDGES = 1600000
NUM_BASIS = 32
CUTOFF = 8.0

NW = 32
PER_W = N_EDGES // NW
CH = 2000
NCHUNK = PER_W // CH
GRP = CH // 16

def _sc_d2_kernel(px_hbm, py_hbm, pz_hbm, src_hbm, dst_hbm, out_hbm,
                  xt, yt, sidx0, sidx1, didx0, didx1, outv0, outv1,
                  sem_i0, sem_i1, sem_b0, sem_b1, sem_s):
    sid = lax.axis_index("s")
    cid = lax.axis_index("c")
    wid = sid * 2 + cid
    base = wid * PER_W
    sidx = (sidx0, sidx1)
    didx = (didx0, didx1)
    outv = (outv0, outv1)
    sem_i = (sem_i0, sem_i1)
    sem_b = (sem_b0, sem_b1)

    def start_idx(c):
        off = base + c * CH
        return (
            pltpu.async_copy(src_hbm.at[pl.ds(off, CH)], sidx[c % 2],
                             sem_i[c % 2]),
            pltpu.async_copy(dst_hbm.at[pl.ds(off, CH)], didx[c % 2],
                             sem_i[c % 2]),
        )

    stage_cps = [pltpu.async_copy(px_hbm, xt, sem_s),
                 pltpu.async_copy(py_hbm, yt, sem_s)]

    out_cps = [None, None]
    idx_cp = start_idx(0)
    for c in range(NCHUNK):
        b = c % 2
        nxt_cp = start_idx(c + 1) if c + 1 < NCHUNK else None
        for cp in idx_cp:
            cp.wait()
        if stage_cps is not None:
            for cp in stage_cps:
                cp.wait()
            stage_cps = None
        if out_cps[b] is not None:
            out_cps[b].wait()
        sb, db, ob = sidx[b], didx[b], outv[b]

        @plsc.parallel_loop(0, GRP, unroll=5)
        def _(g, sb=sb, db=db, ob=ob):
            sl = pl.ds(g * 16, 16)
            si = sb[sl]
            di = db[sl]
            vx = plsc.load_gather(xt, [si]) - plsc.load_gather(xt, [di])
            vy = plsc.load_gather(yt, [si]) - plsc.load_gather(yt, [di])
            ob[sl] = vx * vx + vy * vy

        off = base + c * CH
        out_cps[b] = pltpu.async_copy(ob, out_hbm.at[pl.ds(off, CH)],
                                      sem_b[b])
        idx_cp = nxt_cp
    for cp in out_cps:
        if cp is not None:
            cp.wait()

    stage_cps = [pltpu.async_copy(pz_hbm, xt, sem_s)]

    in_cps = [None, None]
    out_cps = [None, None]
    idx_cp = start_idx(0)
    in_cps[0] = pltpu.async_copy(out_hbm.at[pl.ds(base, CH)], outv0, sem_b0)
    for c in range(NCHUNK):
        b = c % 2
        if c + 1 < NCHUNK:
            nxt_cp = start_idx(c + 1)
            nb = (c + 1) % 2
            if out_cps[nb] is not None:
                out_cps[nb].wait()
            in_cps[nb] = pltpu.async_copy(
                out_hbm.at[pl.ds(base + (c + 1) * CH, CH)], outv[nb],
                sem_b[nb])
        else:
            nxt_cp = None
        for cp in idx_cp:
            cp.wait()
        if stage_cps is not None:
            for cp in stage_cps:
                cp.wait()
            stage_cps = None
        in_cps[b].wait()
        sb, db, ob = sidx[b], didx[b], outv[b]

        @plsc.parallel_loop(0, GRP, unroll=5)
        def _(g, sb=sb, db=db, ob=ob):
            sl = pl.ds(g * 16, 16)
            vz = plsc.load_gather(xt, [sb[sl]]) - plsc.load_gather(xt, [db[sl]])
            ob[sl] = ob[sl] + vz * vz

        off = base + c * CH
        out_cps[b] = pltpu.async_copy(ob, out_hbm.at[pl.ds(off, CH)],
                                      sem_b[b])
        idx_cp = nxt_cp
    for cp in out_cps:
        if cp is not None:
            cp.wait()


@jax.jit
def _sc_d2(px, py, pz, src, dst):
    mesh = plsc.VectorSubcoreMesh(core_axis_name="c", subcore_axis_name="s")
    f = functools.partial(
        pl.kernel,
        mesh=mesh,
        compiler_params=pltpu.CompilerParams(needs_layout_passes=False),
        out_type=jax.ShapeDtypeStruct((N_EDGES,), jnp.float32),
        scratch_types=[
            pltpu.VMEM((N_NODES,), jnp.float32),
            pltpu.VMEM((N_NODES,), jnp.float32),
            pltpu.VMEM((CH,), jnp.int32),
            pltpu.VMEM((CH,), jnp.int32),
            pltpu.VMEM((CH,), jnp.int32),
            pltpu.VMEM((CH,), jnp.int32),
            pltpu.VMEM((CH,), jnp.float32),
            pltpu.VMEM((CH,), jnp.float32),
            pltpu.SemaphoreType.DMA,
            pltpu.SemaphoreType.DMA,
            pltpu.SemaphoreType.DMA,
            pltpu.SemaphoreType.DMA,
            pltpu.SemaphoreType.DMA,
        ],
    )(_sc_d2_kernel)
    return f(px, py, pz, src, dst)


_OFFSETS = np.linspace(0.0, CUTOFF, NUM_BASIS, dtype=np.float32)
_SPACING = float(_OFFSETS[1] - _OFFSETS[0])
_COEFF = float(-0.5 / (_OFFSETS[1] - _OFFSETS[0]) ** 2)

DI_BE = 32768


def _deint_kernel(ei_ref, s_ref, d_ref):
    s_ref[...] = ei_ref[0]
    d_ref[...] = ei_ref[1]


@jax.jit
def _deinterleave(ei):
    grid = ((N_EDGES + DI_BE - 1) // DI_BE,)
    return pl.pallas_call(
        _deint_kernel,
        grid=grid,
        in_specs=[pl.BlockSpec((2, DI_BE), lambda i: (0, i))],
        out_specs=[pl.BlockSpec((DI_BE,), lambda i: (i,)),
                   pl.BlockSpec((DI_BE,), lambda i: (i,))],
        out_shape=[jax.ShapeDtypeStruct((N_EDGES,), jnp.int32),
                   jax.ShapeDtypeStruct((N_EDGES,), jnp.int32)],
    )(ei)


TC_BE = 65536


def _tc_expand_kernel(d2_ref, out_ref):
    d2v = d2_ref[...]
    dist = jnp.sqrt(d2v)
    u = dist * (np.pi / CUTOFF)
    fc = 0.5 * (jnp.cos(u) + 1.0)
    fc = jnp.where(dist < CUTOFF, fc, 0.0)
    db = jnp.broadcast_to(dist[None, :], (NUM_BASIS, TC_BE))
    fcb = jnp.broadcast_to(fc[None, :], (NUM_BASIS, TC_BE))
    offs = lax.broadcasted_iota(
        jnp.int32, (NUM_BASIS, 1), 0).astype(jnp.float32) * _SPACING
    offs_bc = jnp.broadcast_to(offs, (NUM_BASIS, TC_BE))
    t = db - offs_bc
    out_ref[...] = jnp.exp(_COEFF * (t * t)) * fcb


@jax.jit
def _tc_expand(d2):
    grid = ((N_EDGES + TC_BE - 1) // TC_BE,)
    out_t = pl.pallas_call(
        _tc_expand_kernel,
        grid=grid,
        in_specs=[pl.BlockSpec((TC_BE,), lambda i: (i,))],
        out_specs=pl.BlockSpec((NUM_BASIS, TC_BE), lambda i: (0, i)),
        out_shape=jax.ShapeDtypeStruct((NUM_BASIS, N_EDGES), jnp.float32),
    )(d2)
    return out_t.T


def kernel(pos, edge_index):
    px = pos[:, 0]
    py = pos[:, 1]
    pz = pos[:, 2]
    src, dst = _deinterleave(edge_index)
    d2 = _sc_d2(px, py, pz, src, dst)
    return _tc_expand(d2)
